# baseline (speedup 1.0000x reference)
import functools

import jax
import jax.numpy as jnp
from jax import lax
from jax.experimental import pallas as pl
from jax.experimental.pallas import tpu as pltpu
from jax.experimental.pallas import tpu_sc as plsc

NUM_EMB = 1000000
DIM = 64
BATCH = 16384
SEQ = 50
NC = 2
NS = 16
NW = NC * NS
RPT = BATCH // NW
CB = 16
NCHUNK = RPT // CB
NBUF = 2
G = NCHUNK // NBUF


def _build():
    mesh = plsc.VectorSubcoreMesh(core_axis_name="c", subcore_axis_name="s")

    @functools.partial(
        pl.kernel,
        mesh=mesh,
        out_type=jax.ShapeDtypeStruct((BATCH, SEQ, DIM), jnp.float32),
        scratch_types=[
            pltpu.VMEM((RPT, SEQ), jnp.int32),
            [pltpu.VMEM((CB, SEQ, DIM), jnp.float32) for _ in range(NBUF)],
            [pltpu.SemaphoreType.DMA for _ in range(NBUF)],
            [pltpu.SemaphoreType.DMA for _ in range(NBUF)],
        ],
        compiler_params=pltpu.CompilerParams(use_tc_tiling_on_sc=False),
    )
    def gather_kernel(ids_hbm, table_hbm, out_hbm, ids_v, bufs, fsems, wsems):
        wid = lax.axis_index("s") * NC + lax.axis_index("c")
        base = wid * RPT

        pltpu.sync_copy(ids_hbm.at[pl.ds(base, RPT)], ids_v)

        def fill(chunk, b):
            for r in range(CB):
                pltpu.async_copy(
                    table_hbm.at[ids_v.at[chunk * CB + r]],
                    bufs[b].at[r],
                    fsems[b],
                )

        def wait_fill(b):
            pltpu.make_async_copy(
                out_hbm.at[pl.ds(0, CB)], bufs[b], fsems[b]
            ).wait()

        def drain(chunk, b):
            pltpu.async_copy(
                bufs[b], out_hbm.at[pl.ds(base + chunk * CB, CB)], wsems[b]
            )

        def wait_drain(b):
            pltpu.make_async_copy(
                bufs[b], out_hbm.at[pl.ds(0, CB)], wsems[b]
            ).wait()

        for b in range(NBUF):
            fill(b, b)

        def outer(g, carry):
            for b in range(NBUF):
                chunk = g * NBUF + b
                wait_fill(b)
                drain(chunk, b)
                wait_drain(b)
                fill(chunk + NBUF, b)
            return carry

        lax.fori_loop(0, G - 1, outer, 0)

        for b in range(NBUF):
            chunk = (G - 1) * NBUF + b
            wait_fill(b)
            drain(chunk, b)
        for b in range(NBUF):
            wait_drain(b)

    return gather_kernel


_gather = _build()


def kernel(token_ids, EmbeddingLayer):
    return _gather(token_ids.astype(jnp.int32), EmbeddingLayer)
